# Initial kernel scaffold; baseline (speedup 1.0000x reference)
import jax
import jax.numpy as jnp
from jax.experimental import pallas as pl


def kernel(x, clusters_idcs, W_gcn1, W_gcn2, W_inst, b_inst, W_head, b_head):
    raise NotImplementedError("write your pallas kernel here")


# speedup vs baseline: 16.8124x; 16.8124x over previous
import jax
import jax.numpy as jnp
from jax import lax
from jax.experimental import pallas as pl

B, N, D, CC, K, M, KNN = 16, 625, 128, 2, 8, 64, 32
NP = 640
INF = 3e38
PADIDX = 1000


def _bag_kernel(x_ref, xt_ref, idc_ref, w1_ref, w2_ref, wi_ref, bi_ref,
                wht_ref, bh_ref, li_ref, lb_ref):
    f32 = jnp.float32
    xb = x_ref[0]
    xt = xt_ref[0]

    sq = jnp.sum(xb * xb, axis=1, keepdims=True)
    sqT = jnp.sum(xt * xt, axis=0, keepdims=True)
    G = jnp.dot(xb, xt, preferred_element_type=f32)
    col = lax.broadcasted_iota(jnp.int32, (NP, NP), 1)
    dist = sq + sqT - 2.0 * G
    dist = jnp.where(col < N, dist, INF)

    def body(_, carry):
        T, cnt = carry
        dm = jnp.where(dist > T, dist, INF)
        m = jnp.min(dm, axis=1, keepdims=True)
        c = jnp.sum((dist == m).astype(f32), axis=1, keepdims=True)
        active = cnt < (KNN + 1.0)
        return jnp.where(active, m, T), cnt + jnp.where(active, c, 0.0)

    T0 = jnp.full((NP, 1), -INF, f32)
    c0 = jnp.zeros((NP, 1), f32)
    T, _ = lax.fori_loop(0, KNN + 1, body, (T0, c0))

    lt = (dist < T).astype(f32)
    eq = (dist == T).astype(f32)
    n_less = jnp.sum(lt, axis=1, keepdims=True)
    n_tie = jnp.maximum(jnp.sum(eq, axis=1, keepdims=True), 1.0)
    w = lt + eq * ((KNN + 1.0 - n_less) / n_tie)

    rm = jnp.min(dist, axis=1, keepdims=True)
    first = jnp.min(jnp.where(dist == rm, col, NP), axis=1, keepdims=True)
    oneh = (col == first).astype(f32)

    agg = jnp.dot(w - oneh, xb, preferred_element_type=f32) * (1.0 / KNN)
    h = jnp.maximum(jnp.dot(xb + agg, w1_ref[...], preferred_element_type=f32), 0.0)
    pg = jnp.dot(h, w2_ref[...], preferred_element_type=f32)
    keep = jax.nn.sigmoid(pg[:, 1:2] - pg[:, 0:1])

    li = jnp.dot(xb, wi_ref[...], preferred_element_type=f32) + bi_ref[0:1, :]
    li_ref[0] = li
    s = li * keep

    idc = idc_ref[0]
    ncol = lax.broadcasted_iota(jnp.int32, (NP, 128), 0)
    bestv = jnp.full((1, 1), -INF, f32)
    bestk = jnp.zeros((1, 1), jnp.int32)
    for k in range(K):
        memb = (ncol == idc[k:k + 1, :])
        anyk = jnp.max(memb.astype(f32), axis=1, keepdims=True) > 0.0
        ck0 = jnp.max(jnp.where(anyk, s[:, 0:1], -INF), axis=0, keepdims=True)
        ck1 = jnp.max(jnp.where(anyk, s[:, 1:2], -INF), axis=0, keepdims=True)
        ck = jnp.maximum(ck0, ck1)
        upd = ck > bestv
        bestv = jnp.where(upd, ck, bestv)
        bestk = jnp.where(upd, jnp.full((1, 1), k, jnp.int32), bestk)

    kio = lax.broadcasted_iota(jnp.int32, (K, 128), 0)
    ind = (kio == bestk).astype(jnp.int32)
    selrow = jnp.sum(idc * ind, axis=0, keepdims=True)
    cntcol = jnp.sum((ncol == selrow).astype(f32), axis=1, keepdims=True)

    pooled = jnp.dot(xt, cntcol, preferred_element_type=f32) * (1.0 / M)
    lb_ref[0] = jnp.dot(wht_ref[...], pooled, preferred_element_type=f32) + bh_ref[...]


def kernel(x, clusters_idcs, W_gcn1, W_gcn2, W_inst, b_inst, W_head, b_head):
    f32 = jnp.float32
    xp = jnp.zeros((B, NP, D), f32).at[:, :N, :].set(x.astype(f32))
    xt = jnp.swapaxes(xp, 1, 2)
    idc = jnp.full((B, K, 128), PADIDX, jnp.int32).at[:, :, :M].set(
        clusters_idcs.astype(jnp.int32))
    w2p = jnp.zeros((D, D), f32).at[:, :CC].set(W_gcn2.astype(f32))
    wip = jnp.zeros((D, D), f32).at[:, :CC].set(W_inst.astype(f32))
    bip = jnp.zeros((8, D), f32).at[0, :CC].set(b_inst.astype(f32))
    whtp = jnp.zeros((D, D), f32).at[:CC, :].set(W_head.astype(f32).T)
    bhp = jnp.zeros((D, 1), f32).at[:CC, 0].set(b_head.astype(f32))

    li, lb = pl.pallas_call(
        _bag_kernel,
        grid=(B,),
        in_specs=[
            pl.BlockSpec((1, NP, D), lambda i: (i, 0, 0)),
            pl.BlockSpec((1, D, NP), lambda i: (i, 0, 0)),
            pl.BlockSpec((1, K, 128), lambda i: (i, 0, 0)),
            pl.BlockSpec((D, D), lambda i: (0, 0)),
            pl.BlockSpec((D, D), lambda i: (0, 0)),
            pl.BlockSpec((D, D), lambda i: (0, 0)),
            pl.BlockSpec((8, D), lambda i: (0, 0)),
            pl.BlockSpec((D, D), lambda i: (0, 0)),
            pl.BlockSpec((D, 1), lambda i: (0, 0)),
        ],
        out_specs=[
            pl.BlockSpec((1, NP, D), lambda i: (i, 0, 0)),
            pl.BlockSpec((1, D, 1), lambda i: (i, 0, 0)),
        ],
        out_shape=[
            jax.ShapeDtypeStruct((B, NP, D), f32),
            jax.ShapeDtypeStruct((B, D, 1), f32),
        ],
    )(xp, xt, idc, W_gcn1.astype(f32), w2p, wip, bip, whtp, bhp)

    return lb[:, :CC, 0], li[:, :N, :CC]


# speedup vs baseline: 31.9540x; 1.9006x over previous
import jax
import jax.numpy as jnp
from jax import lax
from jax.experimental import pallas as pl

B, N, D, CC, K, M, KNN = 16, 625, 128, 2, 8, 64, 32
NP = 640
INF = 3e38
PADIDX = 1000


def _bag_kernel(x_ref, xt_ref, idc_ref, w1_ref, w2_ref, wi_ref, bi_ref,
                wht_ref, bh_ref, li_ref, lb_ref):
    f32 = jnp.float32
    xb = x_ref[0]
    xt = xt_ref[0]

    sq = jnp.sum(xb * xb, axis=1, keepdims=True)
    sqT = jnp.sum(xt * xt, axis=0, keepdims=True)
    G = jnp.dot(xb, xt, preferred_element_type=f32)
    col = lax.broadcasted_iota(jnp.int32, (NP, NP), 1)
    dist = sq + sqT - 2.0 * G
    dist = jnp.where(col < N, dist, INF)

    rm = jnp.min(dist, axis=1, keepdims=True)
    rM = jnp.max(jnp.where(col < N, dist, -INF), axis=1, keepdims=True)

    def body(_, carry):
        lo, hi = carry
        mid = 0.5 * (lo + hi)
        cnt = jnp.sum((dist <= mid).astype(f32), axis=1, keepdims=True)
        pred = cnt < (KNN + 1.0)
        return jnp.where(pred, mid, lo), jnp.where(pred, hi, mid)

    _, hi = lax.fori_loop(0, 16, body, (rm, rM))

    w = (dist <= hi).astype(f32)
    csel = jnp.sum(w, axis=1, keepdims=True)

    first = jnp.min(jnp.where(dist == rm, col, NP), axis=1, keepdims=True)
    oneh = (col == first).astype(f32)

    agg = jnp.dot(w - oneh, xb, preferred_element_type=f32) * (1.0 / (csel - 1.0))
    h = jnp.maximum(jnp.dot(xb + agg, w1_ref[...], preferred_element_type=f32), 0.0)
    pg = jnp.dot(h, w2_ref[...], preferred_element_type=f32)
    keep = jax.nn.sigmoid(pg[:, 1:2] - pg[:, 0:1])

    li = jnp.dot(xb, wi_ref[...], preferred_element_type=f32) + bi_ref[0:1, :]
    li_ref[0] = li
    s = li * keep

    idc = idc_ref[0]
    ncol = lax.broadcasted_iota(jnp.int32, (NP, 128), 0)
    q = jnp.maximum(s[:, 0:1], s[:, 1:2])
    bestv = jnp.full((1, 1), -INF, f32)
    bestk = jnp.zeros((1, 1), jnp.int32)
    for k in range(K):
        memb = (ncol == idc[k:k + 1, :])
        ck = jnp.max(jnp.where(memb, q, -INF), axis=(0, 1), keepdims=True)
        ck = ck[0:1, 0:1]
        upd = ck > bestv
        bestv = jnp.where(upd, ck, bestv)
        bestk = jnp.where(upd, jnp.full((1, 1), k, jnp.int32), bestk)

    kio = lax.broadcasted_iota(jnp.int32, (K, 128), 0)
    ind = (kio == bestk).astype(jnp.int32)
    selrow = jnp.sum(idc * ind, axis=0, keepdims=True)
    cntcol = jnp.sum((ncol == selrow).astype(f32), axis=1, keepdims=True)

    pooled = jnp.dot(xt, cntcol, preferred_element_type=f32) * (1.0 / M)
    lb_ref[0] = jnp.dot(wht_ref[...], pooled, preferred_element_type=f32) + bh_ref[...]


def kernel(x, clusters_idcs, W_gcn1, W_gcn2, W_inst, b_inst, W_head, b_head):
    f32 = jnp.float32
    xp = jnp.zeros((B, NP, D), f32).at[:, :N, :].set(x.astype(f32))
    xt = jnp.swapaxes(xp, 1, 2)
    idc = jnp.full((B, K, 128), PADIDX, jnp.int32).at[:, :, :M].set(
        clusters_idcs.astype(jnp.int32))
    w2p = jnp.zeros((D, D), f32).at[:, :CC].set(W_gcn2.astype(f32))
    wip = jnp.zeros((D, D), f32).at[:, :CC].set(W_inst.astype(f32))
    bip = jnp.zeros((8, D), f32).at[0, :CC].set(b_inst.astype(f32))
    whtp = jnp.zeros((D, D), f32).at[:CC, :].set(W_head.astype(f32).T)
    bhp = jnp.zeros((D, 1), f32).at[:CC, 0].set(b_head.astype(f32))

    li, lb = pl.pallas_call(
        _bag_kernel,
        grid=(B,),
        in_specs=[
            pl.BlockSpec((1, NP, D), lambda i: (i, 0, 0)),
            pl.BlockSpec((1, D, NP), lambda i: (i, 0, 0)),
            pl.BlockSpec((1, K, 128), lambda i: (i, 0, 0)),
            pl.BlockSpec((D, D), lambda i: (0, 0)),
            pl.BlockSpec((D, D), lambda i: (0, 0)),
            pl.BlockSpec((D, D), lambda i: (0, 0)),
            pl.BlockSpec((8, D), lambda i: (0, 0)),
            pl.BlockSpec((D, D), lambda i: (0, 0)),
            pl.BlockSpec((D, 1), lambda i: (0, 0)),
        ],
        out_specs=[
            pl.BlockSpec((1, NP, D), lambda i: (i, 0, 0)),
            pl.BlockSpec((1, D, 1), lambda i: (i, 0, 0)),
        ],
        out_shape=[
            jax.ShapeDtypeStruct((B, NP, D), f32),
            jax.ShapeDtypeStruct((B, D, 1), f32),
        ],
    )(xp, xt, idc, W_gcn1.astype(f32), w2p, wip, bip, whtp, bhp)

    return lb[:, :CC, 0], li[:, :N, :CC]


# speedup vs baseline: 47.8516x; 1.4975x over previous
import jax
import jax.numpy as jnp
from jax import lax
from jax.experimental import pallas as pl

B, N, D, CC, K, M, KNN = 16, 625, 128, 2, 8, 64, 32
NP = 640
INF = 3e38
PADIDX = 1000


def _bag_kernel(x_ref, xt_ref, idct_ref, w1t_ref, w2t_ref, wit_ref, bit_ref,
                wi_ref, bi_ref, wh_ref, bh_ref, li_ref, lb_ref):
    f32 = jnp.float32
    xb = x_ref[0]
    xt = xt_ref[0]

    rowi = lax.broadcasted_iota(jnp.int32, (NP, 1), 0)
    sq = jnp.sum(xb * xb, axis=1, keepdims=True)
    sqm = jnp.where(rowi < N, sq, INF)
    sqT = jnp.sum(xt * xt, axis=0, keepdims=True)
    G = jnp.dot(xb, xt, preferred_element_type=f32)
    dist = sqm + sqT - 2.0 * G

    rm = jnp.min(dist, axis=0, keepdims=True)
    maxsq = jnp.max(jnp.where(rowi < N, sq, -INF), axis=0, keepdims=True)
    hi0 = (jnp.sqrt(sqT) + jnp.sqrt(maxsq)) ** 2 + 1.0

    def body(_, carry):
        lo, hi = carry
        mid = 0.5 * (lo + hi)
        cnt = jnp.sum((dist <= mid).astype(f32), axis=0, keepdims=True)
        pred = cnt < (KNN + 1.0)
        return jnp.where(pred, mid, lo), jnp.where(pred, hi, mid)

    _, hi = lax.fori_loop(0, 16, body, (rm, hi0))

    wm = ((dist <= hi) & (dist != rm)).astype(f32)
    csel = jnp.sum(wm, axis=0, keepdims=True)

    aggT = jnp.dot(xt, wm, preferred_element_type=f32) * (1.0 / csel)
    hT = jnp.maximum(jnp.dot(w1t_ref[...], xt + aggT, preferred_element_type=f32), 0.0)
    pgT = jnp.dot(w2t_ref[...], hT, preferred_element_type=f32)
    keep = jax.nn.sigmoid(pgT[1:2, :] - pgT[0:1, :])

    li = jnp.dot(xb, wi_ref[...], preferred_element_type=f32) + bi_ref[0:1, :]
    li_ref[0] = li
    liT = jnp.dot(wit_ref[...], xt, preferred_element_type=f32) + bit_ref[...]
    sT = liT * keep

    idcT = idct_ref[0]
    nlane = lax.broadcasted_iota(jnp.int32, (1, NP), 1)
    nlane128 = lax.broadcasted_iota(jnp.int32, (128, NP), 1)
    q = jnp.maximum(sT[0:1, :], sT[1:2, :])
    bestv = jnp.full((1, 1), -INF, f32)
    bestk = jnp.zeros((1, 1), jnp.int32)
    for k in range(K):
        memb = (idcT[:, k:k + 1] == nlane128)
        anyk = jnp.max(memb.astype(f32), axis=0, keepdims=True) > 0.0
        ck = jnp.max(jnp.where(anyk, q, -INF), axis=1, keepdims=True)
        upd = ck > bestv
        bestv = jnp.where(upd, ck, bestv)
        bestk = jnp.where(upd, jnp.full((1, 1), k, jnp.int32), bestk)

    kio = lax.broadcasted_iota(jnp.int32, (1, K), 1)
    ind = (kio == bestk).astype(jnp.int32)
    selcol = jnp.sum(idcT * ind, axis=1, keepdims=True)
    cnt = jnp.sum((selcol == nlane128).astype(f32), axis=0, keepdims=True)

    pooled = jnp.dot(cnt, xb, preferred_element_type=f32) * (1.0 / M)
    lb_ref[0] = jnp.dot(pooled, wh_ref[...], preferred_element_type=f32) + bh_ref[0:1, :]


def kernel(x, clusters_idcs, W_gcn1, W_gcn2, W_inst, b_inst, W_head, b_head):
    f32 = jnp.float32
    xp = jnp.zeros((B, NP, D), f32).at[:, :N, :].set(x.astype(f32))
    xt = jnp.swapaxes(xp, 1, 2)
    idct = jnp.full((B, 128, K), PADIDX, jnp.int32).at[:, :M, :].set(
        jnp.swapaxes(clusters_idcs.astype(jnp.int32), 1, 2))
    w1t = W_gcn1.astype(f32).T
    w2t = jnp.zeros((8, D), f32).at[:CC, :].set(W_gcn2.astype(f32).T)
    wit = jnp.zeros((8, D), f32).at[:CC, :].set(W_inst.astype(f32).T)
    bit = jnp.zeros((8, 1), f32).at[:CC, 0].set(b_inst.astype(f32))
    wip = jnp.zeros((D, D), f32).at[:, :CC].set(W_inst.astype(f32))
    bip = jnp.zeros((8, D), f32).at[0, :CC].set(b_inst.astype(f32))
    whp = jnp.zeros((D, D), f32).at[:, :CC].set(W_head.astype(f32))
    bhp = jnp.zeros((8, D), f32).at[0, :CC].set(b_head.astype(f32))

    li, lb = pl.pallas_call(
        _bag_kernel,
        grid=(B,),
        in_specs=[
            pl.BlockSpec((1, NP, D), lambda i: (i, 0, 0)),
            pl.BlockSpec((1, D, NP), lambda i: (i, 0, 0)),
            pl.BlockSpec((1, 128, K), lambda i: (i, 0, 0)),
            pl.BlockSpec((D, D), lambda i: (0, 0)),
            pl.BlockSpec((8, D), lambda i: (0, 0)),
            pl.BlockSpec((8, D), lambda i: (0, 0)),
            pl.BlockSpec((8, 1), lambda i: (0, 0)),
            pl.BlockSpec((D, D), lambda i: (0, 0)),
            pl.BlockSpec((8, D), lambda i: (0, 0)),
            pl.BlockSpec((D, D), lambda i: (0, 0)),
            pl.BlockSpec((8, D), lambda i: (0, 0)),
        ],
        out_specs=[
            pl.BlockSpec((1, NP, D), lambda i: (i, 0, 0)),
            pl.BlockSpec((1, 1, D), lambda i: (i, 0, 0)),
        ],
        out_shape=[
            jax.ShapeDtypeStruct((B, NP, D), f32),
            jax.ShapeDtypeStruct((B, 1, D), f32),
        ],
    )(xp, xt, idct, w1t, w2t, wit, bit, wip, bip, whp, bhp)

    return lb[:, 0, :CC], li[:, :N, :CC]


# speedup vs baseline: 51.5723x; 1.0778x over previous
import jax
import jax.numpy as jnp
from jax import lax
from jax.experimental import pallas as pl

B, N, D, CC, K, M, KNN = 16, 625, 128, 2, 8, 64, 32
NP = 640
INF = 3e38
PADIDX = 1000


def _bag_kernel(x_ref, xt_ref, idct_ref, w1t_ref, w2t_ref, wit_ref, bit_ref,
                wi_ref, bi_ref, wh_ref, bh_ref, li_ref, lb_ref):
    f32 = jnp.float32
    xb = x_ref[0]
    xt = xt_ref[0]

    rowi = lax.broadcasted_iota(jnp.int32, (NP, 1), 0)
    sq = jnp.sum(xb * xb, axis=1, keepdims=True)
    sqm = jnp.where(rowi < N, sq, INF)
    sqT = jnp.sum(xt * xt, axis=0, keepdims=True)
    G = jnp.dot(xb, xt, preferred_element_type=f32)
    dist = sqm + sqT - 2.0 * G

    rm = jnp.min(dist, axis=0, keepdims=True)
    maxsq = jnp.max(jnp.where(rowi < N, sq, -INF), axis=0, keepdims=True)
    hi0 = (jnp.sqrt(sqT) + jnp.sqrt(maxsq)) ** 2 + 1.0

    def body(_, carry):
        lo, hi = carry
        mid = 0.5 * (lo + hi)
        cnt = jnp.sum((dist <= mid).astype(f32), axis=0, keepdims=True)
        pred = cnt < (KNN + 1.0)
        return jnp.where(pred, mid, lo), jnp.where(pred, hi, mid)

    _, hi = lax.fori_loop(0, 13, body, (rm, hi0))

    wm = ((dist <= hi) & (dist != rm)).astype(f32)
    csel = jnp.sum(wm, axis=0, keepdims=True)

    aggT = jnp.dot(xt, wm, preferred_element_type=f32) * (1.0 / csel)
    hT = jnp.maximum(jnp.dot(w1t_ref[...], xt + aggT, preferred_element_type=f32), 0.0)
    pgT = jnp.dot(w2t_ref[...], hT, preferred_element_type=f32)
    keep = jax.nn.sigmoid(pgT[1:2, :] - pgT[0:1, :])

    li = jnp.dot(xb, wi_ref[...], preferred_element_type=f32) + bi_ref[0:1, :]
    li_ref[0] = li
    liT = jnp.dot(wit_ref[...], xt, preferred_element_type=f32) + bit_ref[...]
    sT = liT * keep

    idcT = idct_ref[0]
    nlane = lax.broadcasted_iota(jnp.int32, (1, NP), 1)
    nlane128 = lax.broadcasted_iota(jnp.int32, (128, NP), 1)
    q = jnp.maximum(sT[0:1, :], sT[1:2, :])
    bestv = jnp.full((1, 1), -INF, f32)
    bestk = jnp.zeros((1, 1), jnp.int32)
    for k in range(K):
        memb = (idcT[:, k:k + 1] == nlane128)
        anyk = jnp.max(memb.astype(f32), axis=0, keepdims=True) > 0.0
        ck = jnp.max(jnp.where(anyk, q, -INF), axis=1, keepdims=True)
        upd = ck > bestv
        bestv = jnp.where(upd, ck, bestv)
        bestk = jnp.where(upd, jnp.full((1, 1), k, jnp.int32), bestk)

    kio = lax.broadcasted_iota(jnp.int32, (1, K), 1)
    ind = (kio == bestk).astype(jnp.int32)
    selcol = jnp.sum(idcT * ind, axis=1, keepdims=True)
    cnt = jnp.sum((selcol == nlane128).astype(f32), axis=0, keepdims=True)

    pooled = jnp.dot(cnt, xb, preferred_element_type=f32) * (1.0 / M)
    lb_ref[0] = jnp.dot(pooled, wh_ref[...], preferred_element_type=f32) + bh_ref[0:1, :]


def kernel(x, clusters_idcs, W_gcn1, W_gcn2, W_inst, b_inst, W_head, b_head):
    f32 = jnp.float32
    xp = jnp.zeros((B, NP, D), f32).at[:, :N, :].set(x.astype(f32))
    xt = jnp.swapaxes(xp, 1, 2)
    idct = jnp.full((B, 128, K), PADIDX, jnp.int32).at[:, :M, :].set(
        jnp.swapaxes(clusters_idcs.astype(jnp.int32), 1, 2))
    w1t = W_gcn1.astype(f32).T
    w2t = jnp.zeros((8, D), f32).at[:CC, :].set(W_gcn2.astype(f32).T)
    wit = jnp.zeros((8, D), f32).at[:CC, :].set(W_inst.astype(f32).T)
    bit = jnp.zeros((8, 1), f32).at[:CC, 0].set(b_inst.astype(f32))
    wip = jnp.zeros((D, D), f32).at[:, :CC].set(W_inst.astype(f32))
    bip = jnp.zeros((8, D), f32).at[0, :CC].set(b_inst.astype(f32))
    whp = jnp.zeros((D, D), f32).at[:, :CC].set(W_head.astype(f32))
    bhp = jnp.zeros((8, D), f32).at[0, :CC].set(b_head.astype(f32))

    li, lb = pl.pallas_call(
        _bag_kernel,
        grid=(B,),
        in_specs=[
            pl.BlockSpec((1, NP, D), lambda i: (i, 0, 0)),
            pl.BlockSpec((1, D, NP), lambda i: (i, 0, 0)),
            pl.BlockSpec((1, 128, K), lambda i: (i, 0, 0)),
            pl.BlockSpec((D, D), lambda i: (0, 0)),
            pl.BlockSpec((8, D), lambda i: (0, 0)),
            pl.BlockSpec((8, D), lambda i: (0, 0)),
            pl.BlockSpec((8, 1), lambda i: (0, 0)),
            pl.BlockSpec((D, D), lambda i: (0, 0)),
            pl.BlockSpec((8, D), lambda i: (0, 0)),
            pl.BlockSpec((D, D), lambda i: (0, 0)),
            pl.BlockSpec((8, D), lambda i: (0, 0)),
        ],
        out_specs=[
            pl.BlockSpec((1, NP, D), lambda i: (i, 0, 0)),
            pl.BlockSpec((1, 1, D), lambda i: (i, 0, 0)),
        ],
        out_shape=[
            jax.ShapeDtypeStruct((B, NP, D), f32),
            jax.ShapeDtypeStruct((B, 1, D), f32),
        ],
    )(xp, xt, idct, w1t, w2t, wit, bit, wip, bip, whp, bhp)

    return lb[:, 0, :CC], li[:, :N, :CC]


# speedup vs baseline: 52.8960x; 1.0257x over previous
import jax
import jax.numpy as jnp
from jax import lax
from jax.experimental import pallas as pl

B, N, D, CC, K, M, KNN = 16, 625, 128, 2, 8, 64, 32
NP = 640
INF = 3e38
PADIDX = 1000


NB = 2


def _bag_kernel(x_ref, xt_ref, idct_ref, w1t_ref, w2t_ref, wit_ref, bit_ref,
                wi_ref, bi_ref, wh_ref, bh_ref, li_ref, lb_ref):
    f32 = jnp.float32
    rowi = lax.broadcasted_iota(jnp.int32, (NP, 1), 0)
    nlane128 = lax.broadcasted_iota(jnp.int32, (128, NP), 1)
    kio = lax.broadcasted_iota(jnp.int32, (1, K), 1)

    dists, rms, inits = [], [], []
    for b in range(NB):
        xb = x_ref[b]
        xt = xt_ref[b]
        sq = jnp.sum(xb * xb, axis=1, keepdims=True)
        sqm = jnp.where(rowi < N, sq, INF)
        sqT = jnp.sum(xt * xt, axis=0, keepdims=True)
        G = jnp.dot(xb, xt, preferred_element_type=f32)
        dist = sqm + sqT - 2.0 * G

        rm = jnp.min(dist, axis=0, keepdims=True)
        maxsq = jnp.max(jnp.where(rowi < N, sq, -INF), axis=0, keepdims=True)
        hi0 = (jnp.sqrt(sqT) + jnp.sqrt(maxsq)) ** 2 + 1.0
        dists.append(dist)
        rms.append(rm)
        inits.append((rm, hi0))

    def body(_, carry):
        out = []
        for b in range(NB):
            lo, hi = carry[b]
            mid = 0.5 * (lo + hi)
            cnt = jnp.sum((dists[b] <= mid).astype(f32), axis=0, keepdims=True)
            pred = cnt < (KNN + 1.0)
            out.append((jnp.where(pred, mid, lo), jnp.where(pred, hi, mid)))
        return tuple(out)

    finals = lax.fori_loop(0, 13, body, tuple(inits))

    for b in range(NB):
        xb, xt, dist, rm = x_ref[b], xt_ref[b], dists[b], rms[b]
        hi = finals[b][1]
        wm = ((dist <= hi) & (dist != rm)).astype(f32)
        csel = jnp.sum(wm, axis=0, keepdims=True)

        aggT = jnp.dot(xt, wm, preferred_element_type=f32) * (1.0 / csel)
        hT = jnp.maximum(jnp.dot(w1t_ref[...], xt + aggT, preferred_element_type=f32), 0.0)
        pgT = jnp.dot(w2t_ref[...], hT, preferred_element_type=f32)
        keep = jax.nn.sigmoid(pgT[1:2, :] - pgT[0:1, :])

        li = jnp.dot(xb, wi_ref[...], preferred_element_type=f32) + bi_ref[0:1, :]
        li_ref[b] = li
        liT = jnp.dot(wit_ref[...], xt, preferred_element_type=f32) + bit_ref[...]
        sT = liT * keep

        idcT = idct_ref[b]
        q = jnp.maximum(sT[0:1, :], sT[1:2, :])
        bestv = jnp.full((1, 1), -INF, f32)
        bestk = jnp.zeros((1, 1), jnp.int32)
        for k in range(K):
            memb = (idcT[:, k:k + 1] == nlane128)
            anyk = jnp.max(memb.astype(f32), axis=0, keepdims=True) > 0.0
            ck = jnp.max(jnp.where(anyk, q, -INF), axis=1, keepdims=True)
            upd = ck > bestv
            bestv = jnp.where(upd, ck, bestv)
            bestk = jnp.where(upd, jnp.full((1, 1), k, jnp.int32), bestk)

        ind = (kio == bestk).astype(jnp.int32)
        selcol = jnp.sum(idcT * ind, axis=1, keepdims=True)
        cnt = jnp.sum((selcol == nlane128).astype(f32), axis=0, keepdims=True)

        pooled = jnp.dot(cnt, xb, preferred_element_type=f32) * (1.0 / M)
        lb_ref[b] = jnp.dot(pooled, wh_ref[...], preferred_element_type=f32) + bh_ref[0:1, :]


def kernel(x, clusters_idcs, W_gcn1, W_gcn2, W_inst, b_inst, W_head, b_head):
    f32 = jnp.float32
    xp = jnp.zeros((B, NP, D), f32).at[:, :N, :].set(x.astype(f32))
    xt = jnp.swapaxes(xp, 1, 2)
    idct = jnp.full((B, 128, K), PADIDX, jnp.int32).at[:, :M, :].set(
        jnp.swapaxes(clusters_idcs.astype(jnp.int32), 1, 2))
    w1t = W_gcn1.astype(f32).T
    w2t = jnp.zeros((8, D), f32).at[:CC, :].set(W_gcn2.astype(f32).T)
    wit = jnp.zeros((8, D), f32).at[:CC, :].set(W_inst.astype(f32).T)
    bit = jnp.zeros((8, 1), f32).at[:CC, 0].set(b_inst.astype(f32))
    wip = jnp.zeros((D, D), f32).at[:, :CC].set(W_inst.astype(f32))
    bip = jnp.zeros((8, D), f32).at[0, :CC].set(b_inst.astype(f32))
    whp = jnp.zeros((D, D), f32).at[:, :CC].set(W_head.astype(f32))
    bhp = jnp.zeros((8, D), f32).at[0, :CC].set(b_head.astype(f32))

    li, lb = pl.pallas_call(
        _bag_kernel,
        grid=(B // NB,),
        in_specs=[
            pl.BlockSpec((NB, NP, D), lambda i: (i, 0, 0)),
            pl.BlockSpec((NB, D, NP), lambda i: (i, 0, 0)),
            pl.BlockSpec((NB, 128, K), lambda i: (i, 0, 0)),
            pl.BlockSpec((D, D), lambda i: (0, 0)),
            pl.BlockSpec((8, D), lambda i: (0, 0)),
            pl.BlockSpec((8, D), lambda i: (0, 0)),
            pl.BlockSpec((8, 1), lambda i: (0, 0)),
            pl.BlockSpec((D, D), lambda i: (0, 0)),
            pl.BlockSpec((8, D), lambda i: (0, 0)),
            pl.BlockSpec((D, D), lambda i: (0, 0)),
            pl.BlockSpec((8, D), lambda i: (0, 0)),
        ],
        out_specs=[
            pl.BlockSpec((NB, NP, D), lambda i: (i, 0, 0)),
            pl.BlockSpec((NB, 1, D), lambda i: (i, 0, 0)),
        ],
        out_shape=[
            jax.ShapeDtypeStruct((B, NP, D), f32),
            jax.ShapeDtypeStruct((B, 1, D), f32),
        ],
    )(xp, xt, idct, w1t, w2t, wit, bit, wip, bip, whp, bhp)

    return lb[:, 0, :CC], li[:, :N, :CC]
